# Initial kernel scaffold; baseline (speedup 1.0000x reference)
import jax
import jax.numpy as jnp
from jax.experimental import pallas as pl


def kernel(sender_idx, receiver_idx, sender_table, receiver_table):
    raise NotImplementedError("write your pallas kernel here")


# speedup vs baseline: 1.8160x; 1.8160x over previous
import functools

import jax
import jax.numpy as jnp
from jax import lax
from jax.experimental import pallas as pl
from jax.experimental.pallas import tpu as pltpu
from jax.experimental.pallas import tpu_sc as plsc

DIM = 128
BATCH = 16384

_info = plsc.get_sparse_core_info()
_NC = _info.num_cores
_NS = _info.num_subcores
_NW = _NC * _NS
_CHUNK = 128
_ROWS = BATCH // _CHUNK
_CPW = _ROWS // _NW
_GROUPS = _CHUNK // 16

_mesh = plsc.VectorSubcoreMesh(core_axis_name="c", subcore_axis_name="s")


@functools.partial(
    pl.kernel,
    mesh=_mesh,
    compiler_params=pltpu.CompilerParams(
        needs_layout_passes=False, use_tc_tiling_on_sc=False),
    out_type=jax.ShapeDtypeStruct((_ROWS, _CHUNK), jnp.float32),
    scratch_types=[
        pltpu.VMEM((_CPW, _CHUNK), jnp.int32),
        pltpu.VMEM((_CPW, _CHUNK), jnp.int32),
        pltpu.VMEM((_CHUNK, DIM), jnp.float32),
        pltpu.VMEM((_CHUNK, DIM), jnp.float32),
        pltpu.VMEM((_CHUNK, DIM), jnp.float32),
        pltpu.VMEM((_CHUNK, DIM), jnp.float32),
        pltpu.VMEM((_CPW, _CHUNK), jnp.float32),
        pltpu.SemaphoreType.DMA,
        pltpu.SemaphoreType.DMA,
    ],
)
def _dne_sc(sidx_hbm, ridx_hbm, stab_hbm, rtab_hbm, out_hbm,
            sidx_v, ridx_v, srows0, srows1, rrows0, rrows1, out_v,
            sem0, sem1):
    wid = lax.axis_index("s") * _NC + lax.axis_index("c")
    base = wid * _CPW
    pltpu.sync_copy(sidx_hbm.at[pl.ds(base, _CPW)], sidx_v)
    pltpu.sync_copy(ridx_hbm.at[pl.ds(base, _CPW)], ridx_v)

    sbuf = (srows0, srows1)
    rbuf = (rrows0, rrows1)
    sems = (sem0, sem1)

    def start(c):
        p = c & 1
        cs = pltpu.async_copy(stab_hbm.at[sidx_v.at[c]], sbuf[p], sems[p])
        cr = pltpu.async_copy(rtab_hbm.at[ridx_v.at[c]], rbuf[p], sems[p])
        return (cs, cr)

    lane = lax.iota(jnp.int32, 16)
    pending = start(0)
    for c in range(_CPW):
        p = c & 1
        pending[0].wait()
        pending[1].wait()
        if c + 1 < _CPW:
            pending = start(c + 1)
        sv_ref = sbuf[p]
        rv_ref = rbuf[p]
        for g in range(_GROUPS):
            rows = lane + (g * 16)

            def body(j, carry, rows=rows, sv_ref=sv_ref, rv_ref=rv_ref):
                del j
                col, dot, ns, nr = carry
                sv = plsc.load_gather(sv_ref, [rows, col])
                rv = plsc.load_gather(rv_ref, [rows, col])
                col = (col + 1) & 127
                return (col, dot + sv * rv, ns + sv * sv, nr + rv * rv)

            z = jnp.zeros((16,), jnp.float32)
            _, dot, ns, nr = lax.fori_loop(0, DIM, body, (lane, z, z, z),
                                           unroll=8)
            x = ns * nr
            i = plsc.bitcast(x, jnp.int32)
            i = jnp.int32(0x5F3759DF) - (i >> 1)
            y = plsc.bitcast(i, jnp.float32)
            for _ in range(3):
                y = y * (1.5 - 0.5 * x * y * y)
            out_v[c, pl.ds(g * 16, 16)] = dot * y

    pltpu.sync_copy(out_v, out_hbm.at[pl.ds(base, _CPW)])


def kernel(sender_idx, receiver_idx, sender_table, receiver_table):
    sidx = sender_idx.astype(jnp.int32).reshape(_ROWS, _CHUNK)
    ridx = receiver_idx.astype(jnp.int32).reshape(_ROWS, _CHUNK)
    out = _dne_sc(sidx, ridx, sender_table, receiver_table)
    return out.reshape(BATCH)


# speedup vs baseline: 2.1542x; 1.1862x over previous
import functools

import jax
import jax.numpy as jnp
from jax import lax
from jax.experimental import pallas as pl
from jax.experimental.pallas import tpu as pltpu
from jax.experimental.pallas import tpu_sc as plsc

DIM = 128
BATCH = 16384

_info = plsc.get_sparse_core_info()
_NC = _info.num_cores
_NS = _info.num_subcores
_NW = _NC * _NS
_CHUNK = 128
_ROWS = BATCH // _CHUNK
_CPW = _ROWS // _NW
_GROUPS = _CHUNK // 16

_mesh = plsc.VectorSubcoreMesh(core_axis_name="c", subcore_axis_name="s")


@functools.partial(
    pl.kernel,
    mesh=_mesh,
    compiler_params=pltpu.CompilerParams(
        needs_layout_passes=False, use_tc_tiling_on_sc=False),
    out_type=jax.ShapeDtypeStruct((_ROWS, _CHUNK), jnp.float32),
    scratch_types=[
        pltpu.VMEM((_CPW, _CHUNK), jnp.int32),
        pltpu.VMEM((_CPW, _CHUNK), jnp.int32),
        pltpu.VMEM((_CHUNK, DIM), jnp.float32),
        pltpu.VMEM((_CHUNK, DIM), jnp.float32),
        pltpu.VMEM((_CHUNK, DIM), jnp.float32),
        pltpu.VMEM((_CHUNK, DIM), jnp.float32),
        pltpu.VMEM((_CPW, _CHUNK), jnp.float32),
        pltpu.SemaphoreType.DMA,
        pltpu.SemaphoreType.DMA,
    ],
)
def _dne_sc(sidx_hbm, ridx_hbm, stab_hbm, rtab_hbm, out_hbm,
            sidx_v, ridx_v, srows0, srows1, rrows0, rrows1, out_v,
            sem0, sem1):
    wid = lax.axis_index("s") * _NC + lax.axis_index("c")
    base = wid * _CPW
    pltpu.sync_copy(sidx_hbm.at[pl.ds(base, _CPW)], sidx_v)
    pltpu.sync_copy(ridx_hbm.at[pl.ds(base, _CPW)], ridx_v)

    sbuf = (srows0, srows1)
    rbuf = (rrows0, rrows1)
    sems = (sem0, sem1)

    def start(c):
        p = c & 1
        cs = pltpu.async_copy(stab_hbm.at[sidx_v.at[c]], sbuf[p], sems[p])
        cr = pltpu.async_copy(rtab_hbm.at[ridx_v.at[c]], rbuf[p], sems[p])
        return (cs, cr)

    lane = lax.iota(jnp.int32, 16)
    pending = start(0)
    for c in range(_CPW):
        p = c & 1
        pending[0].wait()
        pending[1].wait()
        if c + 1 < _CPW:
            pending = start(c + 1)
        sv_ref = sbuf[p]
        rv_ref = rbuf[p]

        def gbody(g, _, c=c, sv_ref=sv_ref, rv_ref=rv_ref):
            rows = lane + g * 16

            def body(j, carry):
                del j
                col, dot, ns, nr = carry
                sv = plsc.load_gather(sv_ref, [rows, col])
                rv = plsc.load_gather(rv_ref, [rows, col])
                col = (col + 1) & 127
                return (col, dot + sv * rv, ns + sv * sv, nr + rv * rv)

            z = jnp.zeros((16,), jnp.float32)
            _, dot, ns, nr = lax.fori_loop(0, DIM, body, (lane, z, z, z),
                                           unroll=8)
            x = ns * nr
            i = plsc.bitcast(x, jnp.int32)
            i = jnp.int32(0x5F3759DF) - (i >> 1)
            y = plsc.bitcast(i, jnp.float32)
            for _ in range(3):
                y = y * (1.5 - 0.5 * x * y * y)
            out_v[c, pl.ds(g * 16, 16)] = dot * y
            return 0

        lax.fori_loop(0, _GROUPS, gbody, 0)

    pltpu.sync_copy(out_v, out_hbm.at[pl.ds(base, _CPW)])


def kernel(sender_idx, receiver_idx, sender_table, receiver_table):
    sidx = sender_idx.astype(jnp.int32).reshape(_ROWS, _CHUNK)
    ridx = receiver_idx.astype(jnp.int32).reshape(_ROWS, _CHUNK)
    out = _dne_sc(sidx, ridx, sender_table, receiver_table)
    return out.reshape(BATCH)
